# baseline (speedup 1.0000x reference)
import jax
import jax.numpy as jnp
from jax import lax
from jax.experimental import pallas as pl
from jax.experimental.pallas import tpu as pltpu

_EPS = 1e-5
_VMEM_LIMIT = 48 * 1024 * 1024


def _make_conv1_kernel(H, W, Cin, C1, TH):
    T = H // TH

    def _body(x_ref, w_ref, y_ref, s_ref, ss_ref):
        zc = jnp.zeros((W + 2, C1), jnp.bfloat16)
        y_ref[0, 0, :, :] = zc
        y_ref[0, H + 1, :, :] = zc
        zr = jnp.zeros((H + 2, C1), jnp.bfloat16)
        y_ref[0, :, 0, :] = zr
        y_ref[0, :, W + 1, :] = zr

        w = w_ref[...]
        s = jnp.zeros((1, C1), jnp.float32)
        ss = jnp.zeros((1, C1), jnp.float32)
        for t in range(T):
            r0 = t * TH
            slabs = [
                x_ref[0, r0 + ki:r0 + ki + TH, kj:kj + W, :].reshape(TH * W, Cin)
                for ki in range(3) for kj in range(3)
            ]
            patch = jnp.concatenate(slabs, axis=1)
            acc = jnp.dot(patch, w, preferred_element_type=jnp.float32)
            s = s + jnp.sum(acc, axis=0, keepdims=True)
            ss = ss + jnp.sum(acc * acc, axis=0, keepdims=True)
            y_ref[0, 1 + r0:1 + r0 + TH, 1:1 + W, :] = (
                acc.reshape(TH, W, C1).astype(jnp.bfloat16))
        s_ref[0] = s
        ss_ref[0] = ss

    return _body


def _make_conv2_kernel(H, W, C1, C2, TH, n_rows_total):
    T = H // TH
    inv_count = 1.0 / float(n_rows_total)

    def _body(y1_ref, w_ref, s1_ref, ss1_ref, g1_ref, b1_ref,
              y2_ref, s2_ref, ss2_ref, z_ref):
        s_tot = jnp.sum(s1_ref[...], axis=0)
        ss_tot = jnp.sum(ss1_ref[...], axis=0)
        mean = s_tot * inv_count
        var = ss_tot * inv_count - mean * mean
        inv = lax.rsqrt(var + _EPS)
        scale = (g1_ref[...] * inv).reshape(1, 1, C1)
        shift = (b1_ref[...] - mean * g1_ref[...] * inv).reshape(1, 1, C1)

        CH = 6 if (H + 2) % 6 == 0 else 2
        for r in range(0, H + 2, CH):
            yv = y1_ref[0, r:r + CH, :, :].astype(jnp.float32)
            z_ref[r:r + CH, :, :] = jnp.maximum(
                yv * scale + shift, 0.0).astype(jnp.bfloat16)
        zc = jnp.zeros((W + 2, C1), jnp.bfloat16)
        z_ref[0, :, :] = zc
        z_ref[H + 1, :, :] = zc
        zr = jnp.zeros((H + 2, C1), jnp.bfloat16)
        z_ref[:, 0, :] = zr
        z_ref[:, W + 1, :] = zr

        w = w_ref[...]
        s = jnp.zeros((1, C2), jnp.float32)
        ss = jnp.zeros((1, C2), jnp.float32)
        for t in range(T):
            r0 = t * TH
            slabs = [
                z_ref[r0 + ki:r0 + ki + TH, kj:kj + W, :].reshape(TH * W, C1)
                for ki in range(3) for kj in range(3)
            ]
            patch = jnp.concatenate(slabs, axis=1)
            acc = jnp.dot(patch, w, preferred_element_type=jnp.float32)
            s = s + jnp.sum(acc, axis=0, keepdims=True)
            ss = ss + jnp.sum(acc * acc, axis=0, keepdims=True)
            y2_ref[0, r0 * W:(r0 + TH) * W, :] = acc.astype(jnp.bfloat16)
        s2_ref[0] = s
        ss2_ref[0] = ss

    return _body


def _make_bn2_apply_kernel(C2, n_rows_total):
    inv_count = 1.0 / float(n_rows_total)

    def _body(y2_ref, s2_ref, ss2_ref, g2_ref, b2_ref, o_ref):
        s_tot = jnp.sum(s2_ref[...], axis=0)
        ss_tot = jnp.sum(ss2_ref[...], axis=0)
        mean = s_tot * inv_count
        var = ss_tot * inv_count - mean * mean
        inv = lax.rsqrt(var + _EPS)
        scale = g2_ref[...] * inv
        shift = b2_ref[...] - mean * scale
        o_ref[...] = jnp.maximum(
            y2_ref[...].astype(jnp.float32) * scale + shift,
            0.0).astype(jnp.bfloat16)

    return _body


def kernel(x_nchw, w1, b1, g1, beta1, w2, b2, g2, beta2):
    del b1, b2
    N, Cin, H, W = x_nchw.shape
    C1, C2 = w1.shape[0], w2.shape[0]
    M = N * H * W
    TH = 16 if H % 16 == 0 else (8 if H % 8 == 0 else H)

    f32 = jnp.float32
    x_nhwc = jnp.transpose(x_nchw, (0, 2, 3, 1))
    x_pad = jnp.pad(x_nhwc, ((0, 0), (1, 1), (1, 1), (0, 0))).astype(jnp.bfloat16)
    w1t = jnp.transpose(w1, (2, 3, 1, 0)).reshape(9 * Cin, C1).astype(jnp.bfloat16)
    w2t = jnp.transpose(w2, (2, 3, 1, 0)).reshape(9 * C1, C2).astype(jnp.bfloat16)
    g1r = g1.reshape(1, C1).astype(f32)
    b1r = beta1.reshape(1, C1).astype(f32)
    g2r = g2.reshape(1, C2).astype(f32)
    b2r = beta2.reshape(1, C2).astype(f32)

    y1p, s1, ss1 = pl.pallas_call(
        _make_conv1_kernel(H, W, Cin, C1, TH),
        out_shape=(jax.ShapeDtypeStruct((N, H + 2, W + 2, C1), jnp.bfloat16),
                   jax.ShapeDtypeStruct((N, 1, C1), f32),
                   jax.ShapeDtypeStruct((N, 1, C1), f32)),
        grid_spec=pltpu.PrefetchScalarGridSpec(
            num_scalar_prefetch=0,
            grid=(N,),
            in_specs=[
                pl.BlockSpec((1, H + 2, W + 2, Cin), lambda n: (n, 0, 0, 0)),
                pl.BlockSpec((9 * Cin, C1), lambda n: (0, 0)),
            ],
            out_specs=(pl.BlockSpec((1, H + 2, W + 2, C1), lambda n: (n, 0, 0, 0)),
                       pl.BlockSpec((1, 1, C1), lambda n: (n, 0, 0)),
                       pl.BlockSpec((1, 1, C1), lambda n: (n, 0, 0))),
        ),
        compiler_params=pltpu.CompilerParams(
            dimension_semantics=("parallel",),
            vmem_limit_bytes=_VMEM_LIMIT,
        ),
    )(x_pad, w1t)

    y2, s2, ss2 = pl.pallas_call(
        _make_conv2_kernel(H, W, C1, C2, TH, M),
        out_shape=(jax.ShapeDtypeStruct((N, H * W, C2), jnp.bfloat16),
                   jax.ShapeDtypeStruct((N, 1, C2), f32),
                   jax.ShapeDtypeStruct((N, 1, C2), f32)),
        grid_spec=pltpu.PrefetchScalarGridSpec(
            num_scalar_prefetch=0,
            grid=(N,),
            in_specs=[
                pl.BlockSpec((1, H + 2, W + 2, C1), lambda n: (n, 0, 0, 0)),
                pl.BlockSpec((9 * C1, C2), lambda n: (0, 0)),
                pl.BlockSpec((N, 1, C1), lambda n: (0, 0, 0)),
                pl.BlockSpec((N, 1, C1), lambda n: (0, 0, 0)),
                pl.BlockSpec((1, C1), lambda n: (0, 0)),
                pl.BlockSpec((1, C1), lambda n: (0, 0)),
            ],
            out_specs=(pl.BlockSpec((1, H * W, C2), lambda n: (n, 0, 0)),
                       pl.BlockSpec((1, 1, C2), lambda n: (n, 0, 0)),
                       pl.BlockSpec((1, 1, C2), lambda n: (n, 0, 0))),
            scratch_shapes=[pltpu.VMEM((H + 2, W + 2, C1), jnp.bfloat16)],
        ),
        compiler_params=pltpu.CompilerParams(
            dimension_semantics=("parallel",),
            vmem_limit_bytes=_VMEM_LIMIT,
        ),
    )(y1p, w2t, s1, ss1, g1r, b1r)

    tm = 2048 if M % 2048 == 0 else M
    out_flat = pl.pallas_call(
        _make_bn2_apply_kernel(C2, M),
        out_shape=jax.ShapeDtypeStruct((M, C2), jnp.bfloat16),
        grid_spec=pltpu.PrefetchScalarGridSpec(
            num_scalar_prefetch=0,
            grid=(M // tm,),
            in_specs=[
                pl.BlockSpec((tm, C2), lambda i: (i, 0)),
                pl.BlockSpec((N, 1, C2), lambda i: (0, 0, 0)),
                pl.BlockSpec((N, 1, C2), lambda i: (0, 0, 0)),
                pl.BlockSpec((1, C2), lambda i: (0, 0)),
                pl.BlockSpec((1, C2), lambda i: (0, 0)),
            ],
            out_specs=pl.BlockSpec((tm, C2), lambda i: (i, 0)),
        ),
        compiler_params=pltpu.CompilerParams(
            dimension_semantics=("parallel",),
            vmem_limit_bytes=_VMEM_LIMIT,
        ),
    )(y2.reshape(M, C2), s2, ss2, g2r, b2r)

    return jnp.transpose(out_flat.reshape(N, H, W, C2),
                         (0, 3, 1, 2)).astype(f32)


# speedup vs baseline: 1.1116x; 1.1116x over previous
import jax
import jax.numpy as jnp
from jax import lax
from jax.experimental import pallas as pl
from jax.experimental.pallas import tpu as pltpu

_EPS = 1e-5
_VMEM_LIMIT = 48 * 1024 * 1024


def _make_conv1_kernel(H, W, Cin, C1, TH):
    T = H // TH

    def _body(x_ref, w_ref, y_ref, s_ref, ss_ref):
        zc = jnp.zeros((W + 2, C1), jnp.bfloat16)
        y_ref[0, 0, :, :] = zc
        y_ref[0, H + 1, :, :] = zc
        zr = jnp.zeros((H + 2, C1), jnp.bfloat16)
        y_ref[0, :, 0, :] = zr
        y_ref[0, :, W + 1, :] = zr

        w = w_ref[...]
        s = jnp.zeros((1, C1), jnp.float32)
        ss = jnp.zeros((1, C1), jnp.float32)
        for t in range(T):
            r0 = t * TH
            slabs = [
                x_ref[0, r0 + ki:r0 + ki + TH, kj:kj + W, :].reshape(TH * W, Cin)
                for ki in range(3) for kj in range(3)
            ]
            patch = jnp.concatenate(slabs, axis=1)
            acc = jnp.dot(patch, w, preferred_element_type=jnp.float32)
            s = s + jnp.sum(acc, axis=0, keepdims=True)
            ss = ss + jnp.sum(acc * acc, axis=0, keepdims=True)
            y_ref[0, 1 + r0:1 + r0 + TH, 1:1 + W, :] = (
                acc.reshape(TH, W, C1).astype(jnp.bfloat16))
        s_ref[0] = s
        ss_ref[0] = ss

    return _body


def _make_conv2_kernel(H, W, C1, C2, TH, n_rows_total):
    T = H // TH
    inv_count = 1.0 / float(n_rows_total)

    def _body(y1_ref, w_ref, s1_ref, ss1_ref, g1_ref, b1_ref,
              y2_ref, s2_ref, ss2_ref, z_ref):
        s_tot = jnp.sum(s1_ref[...], axis=0)
        ss_tot = jnp.sum(ss1_ref[...], axis=0)
        mean = s_tot * inv_count
        var = ss_tot * inv_count - mean * mean
        inv = lax.rsqrt(var + _EPS)
        scale = (g1_ref[...] * inv).reshape(1, 1, C1)
        shift = (b1_ref[...] - mean * g1_ref[...] * inv).reshape(1, 1, C1)

        CH = 6 if (H + 2) % 6 == 0 else 2
        for r in range(0, H + 2, CH):
            yv = y1_ref[0, r:r + CH, :, :].astype(jnp.float32)
            z_ref[r:r + CH, :, :] = jnp.maximum(
                yv * scale + shift, 0.0).astype(jnp.bfloat16)
        zc = jnp.zeros((W + 2, C1), jnp.bfloat16)
        z_ref[0, :, :] = zc
        z_ref[H + 1, :, :] = zc
        zr = jnp.zeros((H + 2, C1), jnp.bfloat16)
        z_ref[:, 0, :] = zr
        z_ref[:, W + 1, :] = zr

        w = w_ref[...]
        s = jnp.zeros((1, C2), jnp.float32)
        ss = jnp.zeros((1, C2), jnp.float32)
        for t in range(T):
            r0 = t * TH
            slabs = [
                z_ref[r0 + ki:r0 + ki + TH, kj:kj + W, :].reshape(TH * W, C1)
                for ki in range(3) for kj in range(3)
            ]
            patch = jnp.concatenate(slabs, axis=1)
            acc = jnp.dot(patch, w, preferred_element_type=jnp.float32)
            s = s + jnp.sum(acc, axis=0, keepdims=True)
            ss = ss + jnp.sum(acc * acc, axis=0, keepdims=True)
            y2_ref[0, r0 * W:(r0 + TH) * W, :] = acc.astype(jnp.bfloat16)
        s2_ref[0] = s
        ss2_ref[0] = ss

    return _body


def _make_bn2_apply_kernel(C2, n_rows_total):
    inv_count = 1.0 / float(n_rows_total)

    def _body(y2_ref, s2_ref, ss2_ref, g2_ref, b2_ref, o_ref):
        s_tot = jnp.sum(s2_ref[...], axis=0)
        ss_tot = jnp.sum(ss2_ref[...], axis=0)
        mean = s_tot * inv_count
        var = ss_tot * inv_count - mean * mean
        inv = lax.rsqrt(var + _EPS)
        scale = g2_ref[...] * inv
        shift = b2_ref[...] - mean * scale
        o_ref[...] = jnp.maximum(
            y2_ref[...].astype(jnp.float32) * scale + shift, 0.0)

    return _body


def kernel(x_nchw, w1, b1, g1, beta1, w2, b2, g2, beta2):
    del b1, b2
    N, Cin, H, W = x_nchw.shape
    C1, C2 = w1.shape[0], w2.shape[0]
    M = N * H * W
    TH = 16 if H % 16 == 0 else (8 if H % 8 == 0 else H)

    f32 = jnp.float32
    x_nhwc = jnp.transpose(x_nchw, (0, 2, 3, 1))
    x_pad = jnp.pad(x_nhwc, ((0, 0), (1, 1), (1, 1), (0, 0))).astype(jnp.bfloat16)
    w1t = jnp.transpose(w1, (2, 3, 1, 0)).reshape(9 * Cin, C1).astype(jnp.bfloat16)
    w2t = jnp.transpose(w2, (2, 3, 1, 0)).reshape(9 * C1, C2).astype(jnp.bfloat16)
    g1r = g1.reshape(1, C1).astype(f32)
    b1r = beta1.reshape(1, C1).astype(f32)
    g2r = g2.reshape(1, C2).astype(f32)
    b2r = beta2.reshape(1, C2).astype(f32)

    y1p, s1, ss1 = pl.pallas_call(
        _make_conv1_kernel(H, W, Cin, C1, TH),
        out_shape=(jax.ShapeDtypeStruct((N, H + 2, W + 2, C1), jnp.bfloat16),
                   jax.ShapeDtypeStruct((N, 1, C1), f32),
                   jax.ShapeDtypeStruct((N, 1, C1), f32)),
        grid_spec=pltpu.PrefetchScalarGridSpec(
            num_scalar_prefetch=0,
            grid=(N,),
            in_specs=[
                pl.BlockSpec((1, H + 2, W + 2, Cin), lambda n: (n, 0, 0, 0)),
                pl.BlockSpec((9 * Cin, C1), lambda n: (0, 0)),
            ],
            out_specs=(pl.BlockSpec((1, H + 2, W + 2, C1), lambda n: (n, 0, 0, 0)),
                       pl.BlockSpec((1, 1, C1), lambda n: (n, 0, 0)),
                       pl.BlockSpec((1, 1, C1), lambda n: (n, 0, 0))),
        ),
        compiler_params=pltpu.CompilerParams(
            dimension_semantics=("parallel",),
            vmem_limit_bytes=_VMEM_LIMIT,
        ),
    )(x_pad, w1t)

    y2, s2, ss2 = pl.pallas_call(
        _make_conv2_kernel(H, W, C1, C2, TH, M),
        out_shape=(jax.ShapeDtypeStruct((N, H * W, C2), jnp.bfloat16),
                   jax.ShapeDtypeStruct((N, 1, C2), f32),
                   jax.ShapeDtypeStruct((N, 1, C2), f32)),
        grid_spec=pltpu.PrefetchScalarGridSpec(
            num_scalar_prefetch=0,
            grid=(N,),
            in_specs=[
                pl.BlockSpec((1, H + 2, W + 2, C1), lambda n: (n, 0, 0, 0)),
                pl.BlockSpec((9 * C1, C2), lambda n: (0, 0)),
                pl.BlockSpec((N, 1, C1), lambda n: (0, 0, 0)),
                pl.BlockSpec((N, 1, C1), lambda n: (0, 0, 0)),
                pl.BlockSpec((1, C1), lambda n: (0, 0)),
                pl.BlockSpec((1, C1), lambda n: (0, 0)),
            ],
            out_specs=(pl.BlockSpec((1, H * W, C2), lambda n: (n, 0, 0)),
                       pl.BlockSpec((1, 1, C2), lambda n: (n, 0, 0)),
                       pl.BlockSpec((1, 1, C2), lambda n: (n, 0, 0))),
            scratch_shapes=[pltpu.VMEM((H + 2, W + 2, C1), jnp.bfloat16)],
        ),
        compiler_params=pltpu.CompilerParams(
            dimension_semantics=("parallel",),
            vmem_limit_bytes=_VMEM_LIMIT,
        ),
    )(y1p, w2t, s1, ss1, g1r, b1r)

    tm = 2048 if M % 2048 == 0 else M
    out_flat = pl.pallas_call(
        _make_bn2_apply_kernel(C2, M),
        out_shape=jax.ShapeDtypeStruct((M, C2), f32),
        grid_spec=pltpu.PrefetchScalarGridSpec(
            num_scalar_prefetch=0,
            grid=(M // tm,),
            in_specs=[
                pl.BlockSpec((tm, C2), lambda i: (i, 0)),
                pl.BlockSpec((N, 1, C2), lambda i: (0, 0, 0)),
                pl.BlockSpec((N, 1, C2), lambda i: (0, 0, 0)),
                pl.BlockSpec((1, C2), lambda i: (0, 0)),
                pl.BlockSpec((1, C2), lambda i: (0, 0)),
            ],
            out_specs=pl.BlockSpec((tm, C2), lambda i: (i, 0)),
        ),
        compiler_params=pltpu.CompilerParams(
            dimension_semantics=("parallel",),
            vmem_limit_bytes=_VMEM_LIMIT,
        ),
    )(y2.reshape(M, C2), s2, ss2, g2r, b2r)

    return jnp.transpose(out_flat.reshape(N, H, W, C2), (0, 3, 1, 2))
